# baseline (speedup 1.0000x reference)
import functools

import jax
import jax.numpy as jnp
from jax import lax
from jax.experimental import pallas as pl
from jax.experimental.pallas import tpu as pltpu
from jax.experimental.pallas import tpu_sc as plsc

B = 16384
DIM = 32
RAW_DIM = 96
OUT_DIM = 64
NC = 2
NS = 16
NW = NC * NS
BPW = B // NW

BV = 2048


def _tc_pack(pt):
    v = pt.shape[1]
    nblk = (v + BV - 1) // BV

    def body(x_ref, o_ref, xt_ref):
        xt_ref[...] = x_ref[...].T
        for s in range(4):
            o_ref[:, s * DIM : (s + 1) * DIM] = xt_ref[s :: 4, :]

    return pl.pallas_call(
        body,
        grid=(nblk,),
        in_specs=[pl.BlockSpec((DIM, BV), lambda i: (0, i))],
        out_specs=pl.BlockSpec((BV // 4, 4 * DIM), lambda i: (i, 0)),
        out_shape=jax.ShapeDtypeStruct((v // 4, 4 * DIM), jnp.float32),
        scratch_shapes=[pltpu.VMEM((BV, DIM), jnp.float32)],
    )(pt)


def _sc_gather3(i4_cat, i4_brand, i4_store, p_cat, p_brand, p_store):
    mesh = plsc.VectorSubcoreMesh(core_axis_name="c", subcore_axis_name="s")

    @functools.partial(
        pl.kernel,
        mesh=mesh,
        out_type=jax.ShapeDtypeStruct((B, 3 * 4 * DIM), jnp.float32),
        scratch_types=[
            pltpu.VMEM((BPW,), jnp.int32),
            pltpu.VMEM((BPW, 4 * DIM), jnp.float32),
            pltpu.SemaphoreType.DMA,
        ],
    )
    def k(ci, bi, si, pc, pb, ps, xo, idx_v, rows_v, sem):
        wid = lax.axis_index("s") * NC + lax.axis_index("c")
        base = wid * BPW
        for t, (i_hbm, t_hbm) in enumerate(
            ((ci, pc), (bi, pb), (si, ps))
        ):
            pltpu.sync_copy(i_hbm.at[pl.ds(base, BPW)], idx_v)
            pltpu.async_copy(t_hbm.at[idx_v], rows_v, sem).wait()
            pltpu.sync_copy(
                rows_v, xo.at[pl.ds(base, BPW), pl.ds(t * 4 * DIM, 4 * DIM)]
            )

    return k(i4_cat, i4_brand, i4_store, p_cat, p_brand, p_store)


BM = 2048


def _tc_project(x, offs, Wt, b2):
    def body(x_ref, o_ref, w_ref, bias_ref, out_ref):
        x = x_ref[...]
        sel = []
        for t in range(3):
            off = o_ref[:, t : t + 1]
            acc = jnp.zeros((BM, DIM), jnp.float32)
            for s in range(4):
                part = x[:, t * 4 * DIM + s * DIM : t * 4 * DIM + (s + 1) * DIM]
                acc = acc + jnp.where(off == s, part, 0.0)
            sel.append(acc)
        xs = jnp.concatenate(sel, axis=1)
        y = jnp.dot(xs, w_ref[...], preferred_element_type=jnp.float32)
        out_ref[...] = jnp.maximum(y + bias_ref[...], 0.0)

    return pl.pallas_call(
        body,
        grid=(B // BM,),
        in_specs=[
            pl.BlockSpec((BM, 3 * 4 * DIM), lambda i: (i, 0)),
            pl.BlockSpec((BM, 4), lambda i: (i, 0)),
            pl.BlockSpec((RAW_DIM, OUT_DIM), lambda i: (0, 0)),
            pl.BlockSpec((1, OUT_DIM), lambda i: (0, 0)),
        ],
        out_specs=pl.BlockSpec((BM, OUT_DIM), lambda i: (i, 0)),
        out_shape=jax.ShapeDtypeStruct((B, OUT_DIM), jnp.float32),
    )(x, offs, Wt, b2)


def kernel(category, brand, store, emb_category, emb_brand, emb_store, W, b):
    ci = category.astype(jnp.int32)
    bi = brand.astype(jnp.int32)
    si = store.astype(jnp.int32)
    p_cat = emb_category.reshape(-1, 4 * DIM)
    p_brand = emb_brand.reshape(-1, 4 * DIM)
    p_store = emb_store.reshape(-1, 4 * DIM)
    x = _sc_gather3(ci >> 2, bi >> 2, si >> 2, p_cat, p_brand, p_store)
    offs = jnp.stack([ci & 3, bi & 3, si & 3, jnp.zeros_like(ci)], axis=1)
    Wt = W.T
    b2 = b.reshape(1, OUT_DIM)
    return _tc_project(x, offs, Wt, b2)


# speedup vs baseline: 1.1043x; 1.1043x over previous
import functools

import jax
import jax.numpy as jnp
from jax import lax
from jax.experimental import pallas as pl
from jax.experimental.pallas import tpu as pltpu
from jax.experimental.pallas import tpu_sc as plsc

B = 16384
DIM = 32
RAW_DIM = 96
OUT_DIM = 64
NC = 2
NS = 16
NW = NC * NS
BPW = B // NW

BV = 2048


def _tc_pack(pt):
    v = pt.shape[1]
    nblk = (v + BV - 1) // BV

    def body(x_ref, o_ref, xt_ref):
        xt_ref[...] = x_ref[...].T
        for s in range(4):
            o_ref[:, s * DIM : (s + 1) * DIM] = xt_ref[s :: 4, :]

    return pl.pallas_call(
        body,
        grid=(nblk,),
        in_specs=[pl.BlockSpec((DIM, BV), lambda i: (0, i))],
        out_specs=pl.BlockSpec((BV // 4, 4 * DIM), lambda i: (i, 0)),
        out_shape=jax.ShapeDtypeStruct((v // 4, 4 * DIM), jnp.float32),
        scratch_shapes=[pltpu.VMEM((BV, DIM), jnp.float32)],
    )(pt)


def _sc_gather3(i4_cat, i4_brand, i4_store, p_cat, p_brand, p_store):
    mesh = plsc.VectorSubcoreMesh(core_axis_name="c", subcore_axis_name="s")

    @functools.partial(
        pl.kernel,
        mesh=mesh,
        out_type=jax.ShapeDtypeStruct((B, 3 * 4 * DIM), jnp.float32),
        scratch_types=[
            pltpu.VMEM((BPW,), jnp.int32),
            pltpu.VMEM((BPW, 4 * DIM), jnp.float32),
            pltpu.SemaphoreType.DMA,
        ],
    )
    def k(ci, bi, si, pc, pb, ps, xo, idx_v, rows_v, sem):
        wid = lax.axis_index("s") * NC + lax.axis_index("c")
        base = wid * BPW
        for t, (i_hbm, t_hbm) in enumerate(
            ((ci, pc), (bi, pb), (si, ps))
        ):
            pltpu.sync_copy(i_hbm.at[pl.ds(base, BPW)], idx_v)
            pltpu.async_copy(t_hbm.at[idx_v], rows_v, sem).wait()
            pltpu.sync_copy(
                rows_v, xo.at[pl.ds(base, BPW), pl.ds(t * 4 * DIM, 4 * DIM)]
            )

    return k(i4_cat, i4_brand, i4_store, p_cat, p_brand, p_store)


BM = 2048


def _tc_project(x, offs, Wt, b2):
    lane_group = 4 * DIM

    def body(x_ref, o_ref, w_ref, bias_ref, out_ref):
        sel = []
        for t in range(3):
            off = jnp.broadcast_to(o_ref[:, t : t + 1], (BM, lane_group))
            grp = lax.broadcasted_iota(jnp.int32, (BM, lane_group), 1) // DIM
            xm = jnp.where(
                grp == off, x_ref[:, t * lane_group : (t + 1) * lane_group], 0.0
            )
            sel.append(
                xm[:, 0:DIM]
                + xm[:, DIM : 2 * DIM]
                + xm[:, 2 * DIM : 3 * DIM]
                + xm[:, 3 * DIM : 4 * DIM]
            )
        xs = jnp.concatenate(sel, axis=1)
        y = jnp.dot(xs, w_ref[...], preferred_element_type=jnp.float32)
        out_ref[...] = jnp.maximum(y + bias_ref[...], 0.0)

    return pl.pallas_call(
        body,
        grid=(B // BM,),
        in_specs=[
            pl.BlockSpec((BM, 3 * 4 * DIM), lambda i: (i, 0)),
            pl.BlockSpec((BM, 4), lambda i: (i, 0)),
            pl.BlockSpec((RAW_DIM, OUT_DIM), lambda i: (0, 0)),
            pl.BlockSpec((1, OUT_DIM), lambda i: (0, 0)),
        ],
        out_specs=pl.BlockSpec((BM, OUT_DIM), lambda i: (i, 0)),
        out_shape=jax.ShapeDtypeStruct((B, OUT_DIM), jnp.float32),
    )(x, offs, Wt, b2)


def kernel(category, brand, store, emb_category, emb_brand, emb_store, W, b):
    ci = category.astype(jnp.int32)
    bi = brand.astype(jnp.int32)
    si = store.astype(jnp.int32)
    p_cat = _tc_pack(emb_category.T)
    p_brand = emb_brand.reshape(-1, 4 * DIM)
    p_store = emb_store.reshape(-1, 4 * DIM)
    x = _sc_gather3(ci >> 2, bi >> 2, si >> 2, p_cat, p_brand, p_store)
    offs = jnp.stack([ci & 3, bi & 3, si & 3, jnp.zeros_like(ci)], axis=1)
    Wt = W.T
    b2 = b.reshape(1, OUT_DIM)
    return _tc_project(x, offs, Wt, b2)


# speedup vs baseline: 1.1047x; 1.0004x over previous
import functools

import jax
import jax.numpy as jnp
from jax import lax
from jax.experimental import pallas as pl
from jax.experimental.pallas import tpu as pltpu
from jax.experimental.pallas import tpu_sc as plsc

B = 16384
DIM = 32
RAW_DIM = 96
OUT_DIM = 64
NC = 2
NS = 16
NW = NC * NS
BPW = B // NW

BV = 2048


def _tc_pack(pt):
    v = pt.shape[1]
    nblk = (v + BV - 1) // BV

    def body(x_ref, o_ref, xt_ref):
        xt_ref[...] = x_ref[...].T
        for s in range(4):
            o_ref[:, s * DIM : (s + 1) * DIM] = xt_ref[s :: 4, :]

    return pl.pallas_call(
        body,
        grid=(nblk,),
        in_specs=[pl.BlockSpec((DIM, BV), lambda i: (0, i))],
        out_specs=pl.BlockSpec((BV // 4, 4 * DIM), lambda i: (i, 0)),
        out_shape=jax.ShapeDtypeStruct((v // 4, 4 * DIM), jnp.float32),
        scratch_shapes=[pltpu.VMEM((BV, DIM), jnp.float32)],
        compiler_params=pltpu.CompilerParams(
            dimension_semantics=("parallel",)
        ),
    )(pt)


def _sc_gather3(i4_cat, i4_brand, i4_store, p_cat, p_brand, p_store):
    mesh = plsc.VectorSubcoreMesh(core_axis_name="c", subcore_axis_name="s")

    @functools.partial(
        pl.kernel,
        mesh=mesh,
        out_type=jax.ShapeDtypeStruct((B, 3 * 4 * DIM), jnp.float32),
        scratch_types=[
            pltpu.VMEM((BPW,), jnp.int32),
            pltpu.VMEM((BPW, 4 * DIM), jnp.float32),
            pltpu.SemaphoreType.DMA,
        ],
    )
    def k(ci, bi, si, pc, pb, ps, xo, idx_v, rows_v, sem):
        wid = lax.axis_index("s") * NC + lax.axis_index("c")
        base = wid * BPW
        for t, (i_hbm, t_hbm) in enumerate(
            ((ci, pc), (bi, pb), (si, ps))
        ):
            pltpu.sync_copy(i_hbm.at[pl.ds(base, BPW)], idx_v)
            pltpu.async_copy(t_hbm.at[idx_v], rows_v, sem).wait()
            pltpu.sync_copy(
                rows_v, xo.at[pl.ds(base, BPW), pl.ds(t * 4 * DIM, 4 * DIM)]
            )

    return k(i4_cat, i4_brand, i4_store, p_cat, p_brand, p_store)


BM = 2048


def _tc_project(x, offs, Wt, b2):
    lane_group = 4 * DIM

    def body(x_ref, o_ref, w_ref, bias_ref, out_ref):
        sel = []
        for t in range(3):
            off = jnp.broadcast_to(o_ref[:, t : t + 1], (BM, lane_group))
            grp = lax.broadcasted_iota(jnp.int32, (BM, lane_group), 1) // DIM
            xm = jnp.where(
                grp == off, x_ref[:, t * lane_group : (t + 1) * lane_group], 0.0
            )
            sel.append(
                xm[:, 0:DIM]
                + xm[:, DIM : 2 * DIM]
                + xm[:, 2 * DIM : 3 * DIM]
                + xm[:, 3 * DIM : 4 * DIM]
            )
        xs = jnp.concatenate(sel, axis=1)
        y = jnp.dot(xs, w_ref[...], preferred_element_type=jnp.float32)
        out_ref[...] = jnp.maximum(y + bias_ref[...], 0.0)

    return pl.pallas_call(
        body,
        grid=(B // BM,),
        in_specs=[
            pl.BlockSpec((BM, 3 * 4 * DIM), lambda i: (i, 0)),
            pl.BlockSpec((BM, 4), lambda i: (i, 0)),
            pl.BlockSpec((RAW_DIM, OUT_DIM), lambda i: (0, 0)),
            pl.BlockSpec((1, OUT_DIM), lambda i: (0, 0)),
        ],
        out_specs=pl.BlockSpec((BM, OUT_DIM), lambda i: (i, 0)),
        out_shape=jax.ShapeDtypeStruct((B, OUT_DIM), jnp.float32),
        compiler_params=pltpu.CompilerParams(
            dimension_semantics=("parallel",)
        ),
    )(x, offs, Wt, b2)


def kernel(category, brand, store, emb_category, emb_brand, emb_store, W, b):
    ci = category.astype(jnp.int32)
    bi = brand.astype(jnp.int32)
    si = store.astype(jnp.int32)
    p_cat = _tc_pack(emb_category.T)
    p_brand = emb_brand.reshape(-1, 4 * DIM)
    p_store = emb_store.reshape(-1, 4 * DIM)
    x = _sc_gather3(ci >> 2, bi >> 2, si >> 2, p_cat, p_brand, p_store)
    offs = jnp.stack([ci & 3, bi & 3, si & 3, jnp.zeros_like(ci)], axis=1)
    Wt = W.T
    b2 = b.reshape(1, OUT_DIM)
    return _tc_project(x, offs, Wt, b2)
